# baseline (speedup 1.0000x reference)
import functools

import jax
import jax.numpy as jnp
from jax import lax
from jax.experimental import pallas as pl
from jax.experimental.pallas import tpu as pltpu
from jax.experimental.pallas import tpu_sc as plsc

_N_NODES = 10000
_N_EDGES = 320000
_D = 128
_W = _D // 2
_NC = 2
_NS = 16
_NW = _NC * _NS
_EPT = _N_EDGES // _NW
_B = 400
_NCHUNK = _EPT // _B
_G = _B // 16


def _sqrt16(x):
    i = plsc.bitcast(x, jnp.int32)
    y = plsc.bitcast(
        lax.shift_right_arithmetic(i, jnp.int32(1)) + jnp.int32(0x1FBD1DF5),
        jnp.float32)
    for _ in range(3):
        y = jnp.float32(0.5) * (y + x / y)
    return y


def _sc_partials(table_w, src_idx, dst_idx, small_dists, weights):
    mesh = plsc.VectorSubcoreMesh(core_axis_name="c", subcore_axis_name="s")

    @functools.partial(
        pl.kernel,
        out_type=jax.ShapeDtypeStruct((_NW, 128), jnp.float32),
        mesh=mesh,
        compiler_params=pltpu.CompilerParams(
            needs_layout_passes=False, use_tc_tiling_on_sc=False),
        scratch_types=[
            pltpu.VMEM((_EPT,), jnp.int32),
            pltpu.VMEM((_EPT,), jnp.int32),
            pltpu.VMEM((2, _B), jnp.float32),
            pltpu.VMEM((2, _B), jnp.float32),
            pltpu.VMEM((2, _B, _W), jnp.int32),
            pltpu.VMEM((2, _B, _W), jnp.int32),
            pltpu.VMEM((128,), jnp.float32),
            pltpu.VMEM((16, 17), jnp.float32),
            pltpu.SemaphoreType.DMA,
            pltpu.SemaphoreType.DMA,
            pltpu.SemaphoreType.DMA,
            pltpu.SemaphoreType.DMA,
            pltpu.SemaphoreType.DMA,
            pltpu.SemaphoreType.DMA,
            pltpu.SemaphoreType.DMA,
            pltpu.SemaphoreType.DMA,
        ],
    )
    def k(table_hbm, src_hbm, dst_hbm, sd_hbm, w_hbm, out_hbm,
          src_v, dst_v, sd_v, w_v, srows, drows, part_v, accs,
          sem_s0, sem_s1, sem_d0, sem_d1,
          sem_sd0, sem_sd1, sem_w0, sem_w1):
        cid = lax.axis_index("c")
        sid = lax.axis_index("s")
        wid = cid * _NS + sid
        base = wid * _EPT
        pltpu.sync_copy(src_hbm.at[pl.ds(base, _EPT)], src_v)
        pltpu.sync_copy(dst_hbm.at[pl.ds(base, _EPT)], dst_v)

        lanes = lax.iota(jnp.int32, _NS)
        sem_s = (sem_s0, sem_s1)
        sem_d = (sem_d0, sem_d1)
        sem_sd = (sem_sd0, sem_sd1)
        sem_w = (sem_w0, sem_w1)

        def chunk_copies(c, ph):
            off = c * _B
            return (
                pltpu.make_async_copy(
                    table_hbm.at[src_v.at[pl.ds(off, _B)]],
                    srows.at[ph], sem_s[ph]),
                pltpu.make_async_copy(
                    table_hbm.at[dst_v.at[pl.ds(off, _B)]],
                    drows.at[ph], sem_d[ph]),
                pltpu.make_async_copy(
                    sd_hbm.at[pl.ds(base + off, _B)], sd_v.at[ph],
                    sem_sd[ph]),
                pltpu.make_async_copy(
                    w_hbm.at[pl.ds(base + off, _B)], w_v.at[ph], sem_w[ph]),
            )

        def start_chunk(c, ph):
            pass

        def wait_chunk(c, ph):
            pass

        def compute_chunk(ph, loss):
            sb = srows.at[ph]
            db = drows.at[ph]


            def group_body(g, loss):
                gbase = g * 16

                for ar in range(16):
                    row = gbase + ar
                    acc = jnp.zeros((16,), jnp.float32)
                    for u in range(_W // 16):
                        su = sb[row, pl.ds(u * 16, 16)]
                        tu = db[row, pl.ds(u * 16, 16)]
                        dv = (plsc.bitcast(su, jnp.bfloat16)
                              - plsc.bitcast(tu, jnp.bfloat16))
                        di = plsc.bitcast(dv, jnp.int32)
                        hi = plsc.bitcast(
                            jnp.bitwise_and(di, jnp.int32(-65536)),
                            jnp.float32)
                        lo = plsc.bitcast(
                            lax.shift_left(di, jnp.int32(16)),
                            jnp.float32)
                        acc = acc + hi * hi
                        acc = acc + lo * lo
                    accs[ar, pl.ds(0, 16)] = acc

                sq = jnp.zeros((16,), jnp.float32)
                for col in range(16):
                    cols = jnp.full((16,), col, jnp.int32)
                    sq = sq + plsc.load_gather(accs, [lanes, cols])
                dist = _sqrt16(sq)
                r = sd_v[ph, pl.ds(gbase, 16)] - dist
                return loss + w_v[ph, pl.ds(gbase, 16)] * r * r

            return lax.fori_loop(0, _G, group_body, loss)

        start_chunk(0, 0)

        def two_body(i, loss):
            c0 = i * 2
            start_chunk(c0 + 1, 1)
            wait_chunk(c0, 0)
            loss = compute_chunk(0, loss)
            start_chunk(c0 + 2, 0)
            wait_chunk(c0 + 1, 1)
            return compute_chunk(1, loss)

        loss = lax.fori_loop(0, (_NCHUNK - 1) // 2, two_body,
                             jnp.zeros((16,), jnp.float32))
        wait_chunk(_NCHUNK - 1, 0)
        loss = compute_chunk(0, loss)

        for j in range(8):
            part_v[pl.ds(j * 16, 16)] = jnp.zeros((16,), jnp.float32)
        part_v[pl.ds(0, 16)] = loss
        pltpu.sync_copy(part_v, out_hbm.at[wid])

    return k(table_w, src_idx, dst_idx, small_dists, weights)


def _tc_finish(parts):
    def body(p_ref, o_ref):
        o_ref[0, 0] = jnp.sum(p_ref[...])

    out = pl.pallas_call(
        body,
        out_shape=jax.ShapeDtypeStruct((1, 1), jnp.float32),
        out_specs=pl.BlockSpec(memory_space=pltpu.SMEM),
    )(parts)
    return out[0, 0]


def kernel(input, edge_indices, small_dists, weights):
    ei = edge_indices.astype(jnp.int32)
    tb = input.astype(jnp.bfloat16).reshape(_N_NODES, _W, 2)
    table_w = jax.lax.bitcast_convert_type(tb, jnp.int32)
    parts = _sc_partials(table_w, ei[:, 0], ei[:, 1], small_dists, weights)
    return _tc_finish(parts)


# speedup vs baseline: 1.0482x; 1.0482x over previous
import functools

import jax
import jax.numpy as jnp
from jax import lax
from jax.experimental import pallas as pl
from jax.experimental.pallas import tpu as pltpu
from jax.experimental.pallas import tpu_sc as plsc

_N_NODES = 10000
_N_EDGES = 320000
_D = 128
_W = _D // 2
_NC = 2
_NS = 16
_NW = _NC * _NS
_EPT = _N_EDGES // _NW
_B = 400
_NCHUNK = _EPT // _B
_G = _B // 16


def _sqrt16(x):
    i = plsc.bitcast(x, jnp.int32)
    y = plsc.bitcast(
        jnp.int32(0x5F3759DF) - lax.shift_right_arithmetic(i, jnp.int32(1)),
        jnp.float32)
    half = jnp.float32(0.5) * x
    for _ in range(3):
        y = y * (jnp.float32(1.5) - half * y * y)
    return x * y


def _sc_partials(table_w, src_idx, dst_idx, small_dists, weights):
    mesh = plsc.VectorSubcoreMesh(core_axis_name="c", subcore_axis_name="s")

    @functools.partial(
        pl.kernel,
        out_type=jax.ShapeDtypeStruct((_NW, 128), jnp.float32),
        mesh=mesh,
        compiler_params=pltpu.CompilerParams(
            needs_layout_passes=False, use_tc_tiling_on_sc=False),
        scratch_types=[
            pltpu.VMEM((_EPT,), jnp.int32),
            pltpu.VMEM((_EPT,), jnp.int32),
            pltpu.VMEM((2, _B), jnp.float32),
            pltpu.VMEM((2, _B), jnp.float32),
            pltpu.VMEM((2, _B, _W), jnp.int32),
            pltpu.VMEM((2, _B, _W), jnp.int32),
            pltpu.VMEM((128,), jnp.float32),
            pltpu.VMEM((16, 17), jnp.float32),
            pltpu.SemaphoreType.DMA,
            pltpu.SemaphoreType.DMA,
            pltpu.SemaphoreType.DMA,
            pltpu.SemaphoreType.DMA,
            pltpu.SemaphoreType.DMA,
            pltpu.SemaphoreType.DMA,
            pltpu.SemaphoreType.DMA,
            pltpu.SemaphoreType.DMA,
        ],
    )
    def k(table_hbm, src_hbm, dst_hbm, sd_hbm, w_hbm, out_hbm,
          src_v, dst_v, sd_v, w_v, srows, drows, part_v, accs,
          sem_s0, sem_s1, sem_d0, sem_d1,
          sem_sd0, sem_sd1, sem_w0, sem_w1):
        cid = lax.axis_index("c")
        sid = lax.axis_index("s")
        wid = cid * _NS + sid
        base = wid * _EPT
        pltpu.sync_copy(src_hbm.at[pl.ds(base, _EPT)], src_v)
        pltpu.sync_copy(dst_hbm.at[pl.ds(base, _EPT)], dst_v)

        lanes = lax.iota(jnp.int32, _NS)
        sem_s = (sem_s0, sem_s1)
        sem_d = (sem_d0, sem_d1)
        sem_sd = (sem_sd0, sem_sd1)
        sem_w = (sem_w0, sem_w1)

        def chunk_copies(c, ph):
            off = c * _B
            return (
                pltpu.make_async_copy(
                    table_hbm.at[src_v.at[pl.ds(off, _B)]],
                    srows.at[ph], sem_s[ph]),
                pltpu.make_async_copy(
                    table_hbm.at[dst_v.at[pl.ds(off, _B)]],
                    drows.at[ph], sem_d[ph]),
                pltpu.make_async_copy(
                    sd_hbm.at[pl.ds(base + off, _B)], sd_v.at[ph],
                    sem_sd[ph]),
                pltpu.make_async_copy(
                    w_hbm.at[pl.ds(base + off, _B)], w_v.at[ph], sem_w[ph]),
            )

        def start_chunk(c, ph):
            for cp in chunk_copies(c, ph):
                cp.start()

        def wait_chunk(c, ph):
            for cp in chunk_copies(c, ph):
                cp.wait()

        def compute_chunk(ph, loss):
            sb = srows.at[ph]
            db = drows.at[ph]


            def group_body(g, loss):
                gbase = g * 16

                for ar in range(16):
                    row = gbase + ar
                    acc_h = jnp.zeros((16,), jnp.float32)
                    acc_l = jnp.zeros((16,), jnp.float32)
                    for u in range(_W // 16):
                        su = sb[row, pl.ds(u * 16, 16)]
                        tu = db[row, pl.ds(u * 16, 16)]
                        dv = (plsc.bitcast(su, jnp.bfloat16)
                              - plsc.bitcast(tu, jnp.bfloat16))
                        di = plsc.bitcast(dv, jnp.int32)
                        hi = plsc.bitcast(
                            jnp.bitwise_and(di, jnp.int32(-65536)),
                            jnp.float32)
                        lo = plsc.bitcast(
                            lax.shift_left(di, jnp.int32(16)),
                            jnp.float32)
                        acc_h = acc_h + hi * hi
                        acc_l = acc_l + lo * lo
                    accs[ar, pl.ds(0, 16)] = acc_h + acc_l

                parts4 = []
                for p4 in range(4):
                    sq4 = plsc.load_gather(
                        accs, [lanes, jnp.full((16,), p4 * 4, jnp.int32)])
                    for col in range(p4 * 4 + 1, p4 * 4 + 4):
                        cols = jnp.full((16,), col, jnp.int32)
                        sq4 = sq4 + plsc.load_gather(accs, [lanes, cols])
                    parts4.append(sq4)
                sq = (parts4[0] + parts4[1]) + (parts4[2] + parts4[3])
                dist = _sqrt16(sq)
                r = sd_v[ph, pl.ds(gbase, 16)] - dist
                return loss + w_v[ph, pl.ds(gbase, 16)] * r * r

            return lax.fori_loop(0, _G, group_body, loss)

        start_chunk(0, 0)

        def two_body(i, loss):
            c0 = i * 2
            start_chunk(c0 + 1, 1)
            wait_chunk(c0, 0)
            loss = compute_chunk(0, loss)
            start_chunk(c0 + 2, 0)
            wait_chunk(c0 + 1, 1)
            return compute_chunk(1, loss)

        loss = lax.fori_loop(0, (_NCHUNK - 1) // 2, two_body,
                             jnp.zeros((16,), jnp.float32))
        wait_chunk(_NCHUNK - 1, 0)
        loss = compute_chunk(0, loss)

        for j in range(8):
            part_v[pl.ds(j * 16, 16)] = jnp.zeros((16,), jnp.float32)
        part_v[pl.ds(0, 16)] = loss
        pltpu.sync_copy(part_v, out_hbm.at[wid])

    return k(table_w, src_idx, dst_idx, small_dists, weights)


def _tc_finish(parts):
    def body(p_ref, o_ref):
        o_ref[0, 0] = jnp.sum(p_ref[...])

    out = pl.pallas_call(
        body,
        out_shape=jax.ShapeDtypeStruct((1, 1), jnp.float32),
        out_specs=pl.BlockSpec(memory_space=pltpu.SMEM),
    )(parts)
    return out[0, 0]


def kernel(input, edge_indices, small_dists, weights):
    ei = edge_indices.astype(jnp.int32)
    tb = input.astype(jnp.bfloat16).reshape(_N_NODES, _W, 2)
    table_w = jax.lax.bitcast_convert_type(tb, jnp.int32)
    parts = _sc_partials(table_w, ei[:, 0], ei[:, 1], small_dists, weights)
    return _tc_finish(parts)


# speedup vs baseline: 1.0994x; 1.0488x over previous
import functools

import jax
import jax.numpy as jnp
from jax import lax
from jax.experimental import pallas as pl
from jax.experimental.pallas import tpu as pltpu
from jax.experimental.pallas import tpu_sc as plsc

_N_NODES = 10000
_N_EDGES = 320000
_D = 128
_W = _D // 2
_NC = 2
_NS = 16
_NW = _NC * _NS
_EPT = _N_EDGES // _NW
_B = 400
_NCHUNK = _EPT // _B
_G = _B // 16


def _sqrt16(x):
    i = plsc.bitcast(x, jnp.int32)
    y = plsc.bitcast(
        jnp.int32(0x5F3759DF) - lax.shift_right_arithmetic(i, jnp.int32(1)),
        jnp.float32)
    half = jnp.float32(0.5) * x
    for _ in range(3):
        y = y * (jnp.float32(1.5) - half * y * y)
    return x * y


def _sc_partials(table_w, src_idx, dst_idx, small_dists, weights):
    mesh = plsc.VectorSubcoreMesh(core_axis_name="c", subcore_axis_name="s")

    @functools.partial(
        pl.kernel,
        out_type=jax.ShapeDtypeStruct((_NW, 128), jnp.float32),
        mesh=mesh,
        compiler_params=pltpu.CompilerParams(
            needs_layout_passes=False, use_tc_tiling_on_sc=False),
        scratch_types=[
            pltpu.VMEM((_EPT,), jnp.int32),
            pltpu.VMEM((_EPT,), jnp.int32),
            pltpu.VMEM((2, _B), jnp.float32),
            pltpu.VMEM((2, _B), jnp.float32),
            pltpu.VMEM((2, _B, _W), jnp.int32),
            pltpu.VMEM((2, _B, _W), jnp.int32),
            pltpu.VMEM((128,), jnp.float32),
            pltpu.VMEM((16, 17), jnp.float32),
            pltpu.SemaphoreType.DMA,
            pltpu.SemaphoreType.DMA,
            pltpu.SemaphoreType.DMA,
            pltpu.SemaphoreType.DMA,
            pltpu.SemaphoreType.DMA,
            pltpu.SemaphoreType.DMA,
            pltpu.SemaphoreType.DMA,
            pltpu.SemaphoreType.DMA,
        ],
    )
    def k(table_hbm, src_hbm, dst_hbm, sd_hbm, w_hbm, out_hbm,
          src_v, dst_v, sd_v, w_v, srows, drows, part_v, accs,
          sem_s0, sem_s1, sem_d0, sem_d1,
          sem_sd0, sem_sd1, sem_w0, sem_w1):
        cid = lax.axis_index("c")
        sid = lax.axis_index("s")
        wid = cid * _NS + sid
        base = wid * _EPT
        pltpu.sync_copy(src_hbm.at[pl.ds(base, _EPT)], src_v)
        pltpu.sync_copy(dst_hbm.at[pl.ds(base, _EPT)], dst_v)

        lanes = lax.iota(jnp.int32, _NS)
        sem_s = (sem_s0, sem_s1)
        sem_d = (sem_d0, sem_d1)
        sem_sd = (sem_sd0, sem_sd1)
        sem_w = (sem_w0, sem_w1)

        def chunk_copies(c, ph):
            off = c * _B
            return (
                pltpu.make_async_copy(
                    table_hbm.at[src_v.at[pl.ds(off, _B)]],
                    srows.at[ph], sem_s[ph]),
                pltpu.make_async_copy(
                    table_hbm.at[dst_v.at[pl.ds(off, _B)]],
                    drows.at[ph], sem_d[ph]),
                pltpu.make_async_copy(
                    sd_hbm.at[pl.ds(base + off, _B)], sd_v.at[ph],
                    sem_sd[ph]),
                pltpu.make_async_copy(
                    w_hbm.at[pl.ds(base + off, _B)], w_v.at[ph], sem_w[ph]),
            )

        def start_chunk(c, ph):
            for cp in chunk_copies(c, ph):
                cp.start()

        def wait_chunk(c, ph):
            for cp in chunk_copies(c, ph):
                cp.wait()

        def compute_chunk(ph, loss):
            sb = srows.at[ph]
            db = drows.at[ph]


            def group_body(g, loss):
                gbase = g * 16

                for ar in range(16):
                    row = gbase + ar
                    pa = []
                    for u in range(_W // 16):
                        su = sb[row, pl.ds(u * 16, 16)]
                        tu = db[row, pl.ds(u * 16, 16)]
                        dv = (plsc.bitcast(su, jnp.bfloat16)
                              - plsc.bitcast(tu, jnp.bfloat16))
                        pa.append(dv * dv)
                    acc_a = pa[0] + pa[1]
                    acc_b = pa[2] + pa[3]
                    ia = plsc.bitcast(acc_a, jnp.int32)
                    ib = plsc.bitcast(acc_b, jnp.int32)
                    hi = (plsc.bitcast(
                        jnp.bitwise_and(ia, jnp.int32(-65536)), jnp.float32)
                        + plsc.bitcast(
                        jnp.bitwise_and(ib, jnp.int32(-65536)), jnp.float32))
                    lo = (plsc.bitcast(
                        lax.shift_left(ia, jnp.int32(16)), jnp.float32)
                        + plsc.bitcast(
                        lax.shift_left(ib, jnp.int32(16)), jnp.float32))
                    accs[ar, pl.ds(0, 16)] = hi + lo

                parts4 = []
                for p4 in range(4):
                    sq4 = plsc.load_gather(
                        accs, [lanes, jnp.full((16,), p4 * 4, jnp.int32)])
                    for col in range(p4 * 4 + 1, p4 * 4 + 4):
                        cols = jnp.full((16,), col, jnp.int32)
                        sq4 = sq4 + plsc.load_gather(accs, [lanes, cols])
                    parts4.append(sq4)
                sq = (parts4[0] + parts4[1]) + (parts4[2] + parts4[3])
                dist = _sqrt16(sq)
                r = sd_v[ph, pl.ds(gbase, 16)] - dist
                return loss + w_v[ph, pl.ds(gbase, 16)] * r * r

            return lax.fori_loop(0, _G, group_body, loss)

        start_chunk(0, 0)

        def two_body(i, loss):
            c0 = i * 2
            start_chunk(c0 + 1, 1)
            wait_chunk(c0, 0)
            loss = compute_chunk(0, loss)
            start_chunk(c0 + 2, 0)
            wait_chunk(c0 + 1, 1)
            return compute_chunk(1, loss)

        loss = lax.fori_loop(0, (_NCHUNK - 1) // 2, two_body,
                             jnp.zeros((16,), jnp.float32))
        wait_chunk(_NCHUNK - 1, 0)
        loss = compute_chunk(0, loss)

        for j in range(8):
            part_v[pl.ds(j * 16, 16)] = jnp.zeros((16,), jnp.float32)
        part_v[pl.ds(0, 16)] = loss
        pltpu.sync_copy(part_v, out_hbm.at[wid])

    return k(table_w, src_idx, dst_idx, small_dists, weights)


def _tc_finish(parts):
    def body(p_ref, o_ref):
        o_ref[0, 0] = jnp.sum(p_ref[...])

    out = pl.pallas_call(
        body,
        out_shape=jax.ShapeDtypeStruct((1, 1), jnp.float32),
        out_specs=pl.BlockSpec(memory_space=pltpu.SMEM),
    )(parts)
    return out[0, 0]


def kernel(input, edge_indices, small_dists, weights):
    ei = edge_indices.astype(jnp.int32)
    tb = input.astype(jnp.bfloat16).reshape(_N_NODES, _W, 2)
    table_w = jax.lax.bitcast_convert_type(tb, jnp.int32)
    parts = _sc_partials(table_w, ei[:, 0], ei[:, 1], small_dists, weights)
    return _tc_finish(parts)


# speedup vs baseline: 1.1397x; 1.0367x over previous
import functools

import jax
import jax.numpy as jnp
from jax import lax
from jax.experimental import pallas as pl
from jax.experimental.pallas import tpu as pltpu
from jax.experimental.pallas import tpu_sc as plsc

_N_NODES = 10000
_N_EDGES = 320000
_D = 128
_W = _D // 2
_NC = 2
_NS = 16
_NW = _NC * _NS
_EPT = _N_EDGES // _NW
_B = 400
_NCHUNK = _EPT // _B
_G = _B // 16


def _sqrt16(x):
    i = plsc.bitcast(x, jnp.int32)
    y = plsc.bitcast(
        jnp.int32(0x5F3759DF) - lax.shift_right_arithmetic(i, jnp.int32(1)),
        jnp.float32)
    half = jnp.float32(0.5) * x
    for _ in range(2):
        y = y * (jnp.float32(1.5) - half * y * y)
    return x * y


def _sc_partials(table_w, src_idx, dst_idx, small_dists, weights):
    mesh = plsc.VectorSubcoreMesh(core_axis_name="c", subcore_axis_name="s")

    @functools.partial(
        pl.kernel,
        out_type=jax.ShapeDtypeStruct((_NW, 128), jnp.float32),
        mesh=mesh,
        compiler_params=pltpu.CompilerParams(
            needs_layout_passes=False, use_tc_tiling_on_sc=False),
        scratch_types=[
            pltpu.VMEM((_EPT,), jnp.int32),
            pltpu.VMEM((_EPT,), jnp.int32),
            pltpu.VMEM((2, _B), jnp.float32),
            pltpu.VMEM((2, _B), jnp.float32),
            pltpu.VMEM((2, _B, _W), jnp.int32),
            pltpu.VMEM((2, _B, _W), jnp.int32),
            pltpu.VMEM((128,), jnp.float32),
            pltpu.VMEM((16, 17), jnp.float32),
            pltpu.VMEM((16, 17), jnp.float32),
            pltpu.SemaphoreType.DMA,
            pltpu.SemaphoreType.DMA,
            pltpu.SemaphoreType.DMA,
            pltpu.SemaphoreType.DMA,
            pltpu.SemaphoreType.DMA,
            pltpu.SemaphoreType.DMA,
            pltpu.SemaphoreType.DMA,
            pltpu.SemaphoreType.DMA,
        ],
    )
    def k(table_hbm, src_hbm, dst_hbm, sd_hbm, w_hbm, out_hbm,
          src_v, dst_v, sd_v, w_v, srows, drows, part_v, accs_a, accs_b,
          sem_s0, sem_s1, sem_d0, sem_d1,
          sem_sd0, sem_sd1, sem_w0, sem_w1):
        cid = lax.axis_index("c")
        sid = lax.axis_index("s")
        wid = cid * _NS + sid
        base = wid * _EPT
        pltpu.sync_copy(src_hbm.at[pl.ds(base, _EPT)], src_v)
        pltpu.sync_copy(dst_hbm.at[pl.ds(base, _EPT)], dst_v)

        lanes = lax.iota(jnp.int32, _NS)
        sem_s = (sem_s0, sem_s1)
        sem_d = (sem_d0, sem_d1)
        sem_sd = (sem_sd0, sem_sd1)
        sem_w = (sem_w0, sem_w1)

        def chunk_copies(c, ph):
            off = c * _B
            return (
                pltpu.make_async_copy(
                    table_hbm.at[src_v.at[pl.ds(off, _B)]],
                    srows.at[ph], sem_s[ph]),
                pltpu.make_async_copy(
                    table_hbm.at[dst_v.at[pl.ds(off, _B)]],
                    drows.at[ph], sem_d[ph]),
                pltpu.make_async_copy(
                    sd_hbm.at[pl.ds(base + off, _B)], sd_v.at[ph],
                    sem_sd[ph]),
                pltpu.make_async_copy(
                    w_hbm.at[pl.ds(base + off, _B)], w_v.at[ph], sem_w[ph]),
            )

        def start_chunk(c, ph):
            for cp in chunk_copies(c, ph):
                cp.start()

        def wait_chunk(c, ph):
            for cp in chunk_copies(c, ph):
                cp.wait()

        def compute_chunk(ph, loss):
            sb = srows.at[ph]
            db = drows.at[ph]

            def edges_block(gbase, acc_ref):
                for ar in range(16):
                    row = gbase + ar
                    pa = []
                    for u in range(_W // 16):
                        su = sb[row, pl.ds(u * 16, 16)]
                        tu = db[row, pl.ds(u * 16, 16)]
                        dv = (plsc.bitcast(su, jnp.bfloat16)
                              - plsc.bitcast(tu, jnp.bfloat16))
                        pa.append(dv * dv)
                    acc_a = pa[0] + pa[1]
                    acc_b = pa[2] + pa[3]
                    ia = plsc.bitcast(acc_a, jnp.int32)
                    ib = plsc.bitcast(acc_b, jnp.int32)
                    hi = (plsc.bitcast(
                        jnp.bitwise_and(ia, jnp.int32(-65536)), jnp.float32)
                        + plsc.bitcast(
                        jnp.bitwise_and(ib, jnp.int32(-65536)), jnp.float32))
                    lo = (plsc.bitcast(
                        lax.shift_left(ia, jnp.int32(16)), jnp.float32)
                        + plsc.bitcast(
                        lax.shift_left(ib, jnp.int32(16)), jnp.float32))
                    acc_ref[ar, pl.ds(0, 16)] = hi + lo

            def tail_block(gbase, acc_ref, loss):
                parts4 = []
                for p4 in range(4):
                    sq4 = plsc.load_gather(
                        acc_ref, [lanes, jnp.full((16,), p4 * 4, jnp.int32)])
                    for col in range(p4 * 4 + 1, p4 * 4 + 4):
                        cols = jnp.full((16,), col, jnp.int32)
                        sq4 = sq4 + plsc.load_gather(acc_ref, [lanes, cols])
                    parts4.append(sq4)
                sq = (parts4[0] + parts4[1]) + (parts4[2] + parts4[3])
                dist = _sqrt16(sq)
                r = sd_v[ph, pl.ds(gbase, 16)] - dist
                return loss + w_v[ph, pl.ds(gbase, 16)] * r * r

            edges_block(0, accs_a)

            def pair_body(i, loss):
                g1 = i * 2 + 1
                edges_block(g1 * 16, accs_b)
                loss = tail_block(i * 2 * 16, accs_a, loss)
                edges_block((g1 + 1) * 16, accs_a)
                return tail_block(g1 * 16, accs_b, loss)

            loss = lax.fori_loop(0, (_G - 1) // 2, pair_body, loss)
            return tail_block((_G - 1) * 16, accs_a, loss)

        start_chunk(0, 0)

        def two_body(i, loss):
            c0 = i * 2
            start_chunk(c0 + 1, 1)
            wait_chunk(c0, 0)
            loss = compute_chunk(0, loss)
            start_chunk(c0 + 2, 0)
            wait_chunk(c0 + 1, 1)
            return compute_chunk(1, loss)

        loss = lax.fori_loop(0, (_NCHUNK - 1) // 2, two_body,
                             jnp.zeros((16,), jnp.float32))
        wait_chunk(_NCHUNK - 1, 0)
        loss = compute_chunk(0, loss)

        for j in range(8):
            part_v[pl.ds(j * 16, 16)] = jnp.zeros((16,), jnp.float32)
        part_v[pl.ds(0, 16)] = loss
        pltpu.sync_copy(part_v, out_hbm.at[wid])

    return k(table_w, src_idx, dst_idx, small_dists, weights)


def _tc_finish(parts):
    def body(p_ref, o_ref):
        o_ref[0, 0] = jnp.sum(p_ref[...])

    out = pl.pallas_call(
        body,
        out_shape=jax.ShapeDtypeStruct((1, 1), jnp.float32),
        out_specs=pl.BlockSpec(memory_space=pltpu.SMEM),
    )(parts)
    return out[0, 0]


def kernel(input, edge_indices, small_dists, weights):
    ei = edge_indices.astype(jnp.int32)
    tb = input.astype(jnp.bfloat16).reshape(_N_NODES, _W, 2)
    table_w = jax.lax.bitcast_convert_type(tb, jnp.int32)
    parts = _sc_partials(table_w, ei[:, 0], ei[:, 1], small_dists, weights)
    return _tc_finish(parts)


# speedup vs baseline: 1.1419x; 1.0019x over previous
import functools

import jax
import jax.numpy as jnp
from jax import lax
from jax.experimental import pallas as pl
from jax.experimental.pallas import tpu as pltpu
from jax.experimental.pallas import tpu_sc as plsc

_N_NODES = 10000
_N_EDGES = 320000
_D = 128
_W = _D // 2
_NC = 2
_NS = 16
_NW = _NC * _NS
_EPT = _N_EDGES // _NW
_B = 400
_NCHUNK = _EPT // _B
_G = _B // 16


def _sqrt16(x):
    i = plsc.bitcast(x, jnp.int32)
    y = plsc.bitcast(
        jnp.int32(0x5F3759DF) - lax.shift_right_arithmetic(i, jnp.int32(1)),
        jnp.float32)
    half = jnp.float32(0.5) * x
    for _ in range(2):
        y = y * (jnp.float32(1.5) - half * y * y)
    return x * y


def _sc_partials(table_w, src_idx, dst_idx, small_dists, weights):
    mesh = plsc.VectorSubcoreMesh(core_axis_name="c", subcore_axis_name="s")

    @functools.partial(
        pl.kernel,
        out_type=jax.ShapeDtypeStruct((_NW, 128), jnp.float32),
        mesh=mesh,
        compiler_params=pltpu.CompilerParams(
            needs_layout_passes=False, use_tc_tiling_on_sc=False),
        scratch_types=[
            pltpu.VMEM((_EPT,), jnp.int32),
            pltpu.VMEM((_EPT,), jnp.int32),
            pltpu.VMEM((2, _B), jnp.float32),
            pltpu.VMEM((2, _B), jnp.float32),
            pltpu.VMEM((2, _B, _W), jnp.int32),
            pltpu.VMEM((2, _B, _W), jnp.int32),
            pltpu.VMEM((128,), jnp.float32),
            pltpu.VMEM((16, 17), jnp.float32),
            pltpu.VMEM((16, 17), jnp.float32),
            pltpu.SemaphoreType.DMA,
            pltpu.SemaphoreType.DMA,
            pltpu.SemaphoreType.DMA,
            pltpu.SemaphoreType.DMA,
            pltpu.SemaphoreType.DMA,
            pltpu.SemaphoreType.DMA,
            pltpu.SemaphoreType.DMA,
            pltpu.SemaphoreType.DMA,
        ],
    )
    def k(table_hbm, src_hbm, dst_hbm, sd_hbm, w_hbm, out_hbm,
          src_v, dst_v, sd_v, w_v, srows, drows, part_v, accs_a, accs_b,
          sem_s0, sem_s1, sem_d0, sem_d1,
          sem_sd0, sem_sd1, sem_w0, sem_w1):
        cid = lax.axis_index("c")
        sid = lax.axis_index("s")
        wid = cid * _NS + sid
        base = wid * _EPT
        pltpu.sync_copy(src_hbm.at[pl.ds(base, _EPT)], src_v)
        pltpu.sync_copy(dst_hbm.at[pl.ds(base, _EPT)], dst_v)

        lanes = lax.iota(jnp.int32, _NS)
        sem_s = (sem_s0, sem_s1)
        sem_d = (sem_d0, sem_d1)
        sem_sd = (sem_sd0, sem_sd1)
        sem_w = (sem_w0, sem_w1)

        def chunk_copies(c, ph):
            off = c * _B
            return (
                pltpu.make_async_copy(
                    table_hbm.at[src_v.at[pl.ds(off, _B)]],
                    srows.at[ph], sem_s[ph]),
                pltpu.make_async_copy(
                    table_hbm.at[dst_v.at[pl.ds(off, _B)]],
                    drows.at[ph], sem_d[ph]),
                pltpu.make_async_copy(
                    sd_hbm.at[pl.ds(base + off, _B)], sd_v.at[ph],
                    sem_sd[ph]),
                pltpu.make_async_copy(
                    w_hbm.at[pl.ds(base + off, _B)], w_v.at[ph], sem_w[ph]),
            )

        def start_chunk(c, ph):
            for cp in chunk_copies(c, ph):
                cp.start()

        def wait_chunk(c, ph):
            for cp in chunk_copies(c, ph):
                cp.wait()

        def compute_chunk(ph, loss):
            sb = srows.at[ph]
            db = drows.at[ph]

            def edges_block(gbase, acc_ref):
                for ar in range(16):
                    row = gbase + ar
                    pa = []
                    for u in range(_W // 16):
                        su = sb[row, pl.ds(u * 16, 16)]
                        tu = db[row, pl.ds(u * 16, 16)]
                        dv = (plsc.bitcast(su, jnp.bfloat16)
                              - plsc.bitcast(tu, jnp.bfloat16))
                        pa.append(dv * dv)
                    acc = (pa[0] + pa[1]) + (pa[2] + pa[3])
                    ia = plsc.bitcast(acc, jnp.int32)
                    hi = plsc.bitcast(
                        jnp.bitwise_and(ia, jnp.int32(-65536)), jnp.float32)
                    lo = plsc.bitcast(
                        lax.shift_left(ia, jnp.int32(16)), jnp.float32)
                    acc_ref[ar, pl.ds(0, 16)] = hi + lo

            def tail_block(gbase, acc_ref, loss):
                parts4 = []
                for p4 in range(4):
                    sq4 = plsc.load_gather(
                        acc_ref, [lanes, jnp.full((16,), p4 * 4, jnp.int32)])
                    for col in range(p4 * 4 + 1, p4 * 4 + 4):
                        cols = jnp.full((16,), col, jnp.int32)
                        sq4 = sq4 + plsc.load_gather(acc_ref, [lanes, cols])
                    parts4.append(sq4)
                sq = (parts4[0] + parts4[1]) + (parts4[2] + parts4[3])
                dist = _sqrt16(sq)
                r = sd_v[ph, pl.ds(gbase, 16)] - dist
                return loss + w_v[ph, pl.ds(gbase, 16)] * r * r

            edges_block(0, accs_a)

            def pair_body(i, loss):
                g1 = i * 2 + 1
                edges_block(g1 * 16, accs_b)
                loss = tail_block(i * 2 * 16, accs_a, loss)
                edges_block((g1 + 1) * 16, accs_a)
                return tail_block(g1 * 16, accs_b, loss)

            loss = lax.fori_loop(0, (_G - 1) // 2, pair_body, loss)
            return tail_block((_G - 1) * 16, accs_a, loss)

        start_chunk(0, 0)

        def two_body(i, loss):
            c0 = i * 2
            start_chunk(c0 + 1, 1)
            wait_chunk(c0, 0)
            loss = compute_chunk(0, loss)
            start_chunk(c0 + 2, 0)
            wait_chunk(c0 + 1, 1)
            return compute_chunk(1, loss)

        loss = lax.fori_loop(0, (_NCHUNK - 1) // 2, two_body,
                             jnp.zeros((16,), jnp.float32))
        wait_chunk(_NCHUNK - 1, 0)
        loss = compute_chunk(0, loss)

        for j in range(8):
            part_v[pl.ds(j * 16, 16)] = jnp.zeros((16,), jnp.float32)
        part_v[pl.ds(0, 16)] = loss
        pltpu.sync_copy(part_v, out_hbm.at[wid])

    return k(table_w, src_idx, dst_idx, small_dists, weights)


def _tc_finish(parts):
    def body(p_ref, o_ref):
        o_ref[0, 0] = jnp.sum(p_ref[...])

    out = pl.pallas_call(
        body,
        out_shape=jax.ShapeDtypeStruct((1, 1), jnp.float32),
        out_specs=pl.BlockSpec(memory_space=pltpu.SMEM),
    )(parts)
    return out[0, 0]


def kernel(input, edge_indices, small_dists, weights):
    ei = edge_indices.astype(jnp.int32)
    tb = input.astype(jnp.bfloat16).reshape(_N_NODES, _W, 2)
    table_w = jax.lax.bitcast_convert_type(tb, jnp.int32)
    parts = _sc_partials(table_w, ei[:, 0], ei[:, 1], small_dists, weights)
    return _tc_finish(parts)
